# baseline (speedup 1.0000x reference)
import functools

import jax
import jax.numpy as jnp
from jax import lax
from jax.experimental import pallas as pl
from jax.experimental.pallas import tpu as pltpu
from jax.experimental.pallas import tpu_sc as plsc

BATCH = 16384
EMB_DIM = 32
LANES = 16
NSUPER = 250000
SROW = 128

_info = plsc.get_sparse_core_info()
_NC, _NS = _info.num_cores, _info.num_subcores
NW = _NC * _NS
B_PER_W = BATCH // NW
CHUNK = 128
N_CHUNKS = B_PER_W // CHUNK
G_PER_CHUNK = CHUNK // LANES


def _body(user_hbm, item_hbm, ue_hbm, ie_hbm, ubias_hbm, ibias_hbm,
          out_hbm, uidx_v, iidx_v, uq_v, iq_v, ubuf_v, ibuf_v,
          ubias_v, ibias_v, out_v, sem, bsem):
    wid = lax.axis_index("s") * _NC + lax.axis_index("c")
    base = wid * B_PER_W

    pltpu.sync_copy(user_hbm.at[pl.ds(base, B_PER_W)], uidx_v)
    pltpu.sync_copy(item_hbm.at[pl.ds(base, B_PER_W)], iidx_v)

    bias_handles = []
    for j in range(N_CHUNKS):
        sl = pl.ds(j * CHUNK, CHUNK)
        bias_handles.append(pltpu.async_copy(
            ubias_hbm.at[0].at[uidx_v.at[sl]], ubias_v.at[sl], bsem))
        bias_handles.append(pltpu.async_copy(
            ibias_hbm.at[0].at[iidx_v.at[sl]], ibias_v.at[sl], bsem))

    def shift(k, _):
        sl = pl.ds(k * LANES, LANES)
        uq_v[sl] = lax.shift_right_logical(uidx_v[sl], 2)
        iq_v[sl] = lax.shift_right_logical(iidx_v[sl], 2)
        return 0

    lax.fori_loop(0, B_PER_W // LANES, shift, 0)

    lane_ids = lax.iota(jnp.int32, LANES)

    def chunk_body(j, _):
        csl = pl.ds(j * CHUNK, CHUNK)
        hu = pltpu.async_copy(ue_hbm.at[uq_v.at[csl]], ubuf_v, sem)
        hi_ = pltpu.async_copy(ie_hbm.at[iq_v.at[csl]], ibuf_v, sem)
        hu.wait()
        hi_.wait()

        def group(gg, _):
            rb = j * CHUNK + gg * LANES
            lb = gg * LANES
            uvec = uidx_v[pl.ds(rb, LANES)]
            ivec = iidx_v[pl.ds(rb, LANES)]
            acc = ubias_v[pl.ds(rb, LANES)] + ibias_v[pl.ds(rb, LANES)]
            dots = jnp.zeros((LANES,), jnp.float32)
            for b in range(LANES):
                uo = (uvec[b] & 3) * 32
                io = (ivec[b] & 3) * 32
                u_lo = ubuf_v[lb + b, pl.ds(uo, LANES)]
                u_hi = ubuf_v[lb + b, pl.ds(uo + LANES, LANES)]
                i_lo = ibuf_v[lb + b, pl.ds(io, LANES)]
                i_hi = ibuf_v[lb + b, pl.ds(io + LANES, LANES)]
                s = jnp.sum(u_lo * i_lo + u_hi * i_hi)
                dots = jnp.where(lane_ids == b, s, dots)
            out_v[pl.ds(rb, LANES)] = acc + dots
            return 0

        lax.fori_loop(0, G_PER_CHUNK, group, 0)
        return 0

    lax.fori_loop(0, N_CHUNKS, chunk_body, 0)

    for h in bias_handles:
        h.wait()
    pltpu.sync_copy(out_v, out_hbm.at[pl.ds(base, B_PER_W)])


@jax.jit
def _run(user, item, ue, ie, ubias_t, ibias_t):
    mesh = plsc.VectorSubcoreMesh(core_axis_name="c", subcore_axis_name="s")
    return pl.kernel(
        _body,
        mesh=mesh,
        out_type=jax.ShapeDtypeStruct((BATCH,), jnp.float32),
        scratch_types=[
            pltpu.VMEM((B_PER_W,), jnp.int32),
            pltpu.VMEM((B_PER_W,), jnp.int32),
            pltpu.VMEM((B_PER_W,), jnp.int32),
            pltpu.VMEM((B_PER_W,), jnp.int32),
            pltpu.VMEM((CHUNK, SROW), jnp.float32),
            pltpu.VMEM((CHUNK, SROW), jnp.float32),
            pltpu.VMEM((B_PER_W,), jnp.float32),
            pltpu.VMEM((B_PER_W,), jnp.float32),
            pltpu.VMEM((B_PER_W,), jnp.float32),
            pltpu.SemaphoreType.DMA,
            pltpu.SemaphoreType.DMA,
        ],
        compiler_params=pltpu.CompilerParams(
            needs_layout_passes=False, use_tc_tiling_on_sc=False),
    )(user, item, ue, ie, ubias_t, ibias_t)


def kernel(user, item, user_embedding, item_embedding, user_bias, item_bias):
    ue = user_embedding.reshape(NSUPER, SROW)
    ie = item_embedding.reshape(NSUPER, SROW)
    return _run(user.astype(jnp.int32), item.astype(jnp.int32), ue, ie,
                user_bias.T, item_bias.T)


# speedup vs baseline: 3.5806x; 3.5806x over previous
import functools

import jax
import jax.numpy as jnp
from jax import lax
from jax.experimental import pallas as pl
from jax.experimental.pallas import tpu as pltpu
from jax.experimental.pallas import tpu_sc as plsc

BATCH = 16384
EMB_DIM = 32
LANES = 16
NUSERS = 1000000
HB = 8

_info = plsc.get_sparse_core_info()
_NC, _NS = _info.num_cores, _info.num_subcores
NW = _NC * _NS
B_PER_W = BATCH // NW
CHUNK = 128
N_CHUNKS = B_PER_W // CHUNK
N_GROUPS = B_PER_W // LANES


def _body(user_hbm, item_hbm, uet_hbm, iet_hbm, ubias_hbm, ibias_hbm,
          out_hbm, uidx_v, iidx_v, ublk_v, iblk_v, ubias_v, ibias_v, out_v,
          sem, bsem):
    wid = lax.axis_index("s") * _NC + lax.axis_index("c")
    base = wid * B_PER_W

    pltpu.sync_copy(user_hbm.at[pl.ds(base, B_PER_W)], uidx_v)
    pltpu.sync_copy(item_hbm.at[pl.ds(base, B_PER_W)], iidx_v)

    bias_handles = []
    for j in range(N_CHUNKS):
        sl = pl.ds(j * CHUNK, CHUNK)
        bias_handles.append(pltpu.async_copy(
            ubias_hbm.at[0].at[uidx_v.at[sl]], ubias_v.at[sl], bsem))
        bias_handles.append(pltpu.async_copy(
            ibias_hbm.at[0].at[iidx_v.at[sl]], ibias_v.at[sl], bsem))

    lane_ids = lax.iota(jnp.int32, LANES)
    hi01 = lane_ids // 8
    hi23 = hi01 + 2
    lo8 = lane_ids % 8

    def group(g, _):
        rb = g * LANES
        uvec = uidx_v[pl.ds(rb, LANES)]
        ivec = iidx_v[pl.ds(rb, LANES)]
        dots = jnp.zeros((LANES,), jnp.float32)
        for half in range(2):
            handles = []
            for b in range(HB):
                l = half * HB + b
                ub = pl.multiple_of((uvec[l] >> 7) * 128, 128)
                ib = pl.multiple_of((ivec[l] >> 7) * 128, 128)
                handles.append(pltpu.async_copy(
                    uet_hbm.at[:, :, pl.ds(ub, 128)], ublk_v.at[b], sem))
                handles.append(pltpu.async_copy(
                    iet_hbm.at[:, :, pl.ds(ib, 128)], iblk_v.at[b], sem))
            for h in handles:
                h.wait()
            for b in range(HB):
                l = half * HB + b
                ucol = jnp.broadcast_to(uvec[l] & 127, (LANES,))
                icol = jnp.broadcast_to(ivec[l] & 127, (LANES,))
                bv = jnp.full((LANES,), b, dtype=jnp.int32)
                u1 = plsc.load_gather(ublk_v, [bv, hi01, lo8, ucol])
                u2 = plsc.load_gather(ublk_v, [bv, hi23, lo8, ucol])
                i1 = plsc.load_gather(iblk_v, [bv, hi01, lo8, icol])
                i2 = plsc.load_gather(iblk_v, [bv, hi23, lo8, icol])
                s = jnp.sum(u1 * i1 + u2 * i2)
                dots = jnp.where(lane_ids == l, s, dots)
        out_v[pl.ds(rb, LANES)] = (dots + ubias_v[pl.ds(rb, LANES)]
                                   + ibias_v[pl.ds(rb, LANES)])
        return 0

    lax.fori_loop(0, N_GROUPS, group, 0)

    for h in bias_handles:
        h.wait()
    pltpu.sync_copy(out_v, out_hbm.at[pl.ds(base, B_PER_W)])


@jax.jit
def _run(user, item, uet, iet, ubias_t, ibias_t):
    mesh = plsc.VectorSubcoreMesh(core_axis_name="c", subcore_axis_name="s")
    return pl.kernel(
        _body,
        mesh=mesh,
        out_type=jax.ShapeDtypeStruct((BATCH,), jnp.float32),
        scratch_types=[
            pltpu.VMEM((B_PER_W,), jnp.int32),
            pltpu.VMEM((B_PER_W,), jnp.int32),
            pltpu.VMEM((HB, 4, 8, 128), jnp.float32),
            pltpu.VMEM((HB, 4, 8, 128), jnp.float32),
            pltpu.VMEM((B_PER_W,), jnp.float32),
            pltpu.VMEM((B_PER_W,), jnp.float32),
            pltpu.VMEM((B_PER_W,), jnp.float32),
            pltpu.SemaphoreType.DMA,
            pltpu.SemaphoreType.DMA,
        ],
        compiler_params=pltpu.CompilerParams(
            needs_layout_passes=False, disable_bounds_checks=True),
    )(user, item, uet, iet, ubias_t, ibias_t)


def kernel(user, item, user_embedding, item_embedding, user_bias, item_bias):
    uet = user_embedding.T.reshape(4, 8, NUSERS)
    iet = item_embedding.T.reshape(4, 8, NUSERS)
    return _run(user.astype(jnp.int32), item.astype(jnp.int32), uet, iet,
                user_bias.T, item_bias.T)


# speedup vs baseline: 3.6521x; 1.0200x over previous
import functools

import jax
import jax.numpy as jnp
from jax import lax
from jax.experimental import pallas as pl
from jax.experimental.pallas import tpu as pltpu
from jax.experimental.pallas import tpu_sc as plsc

BATCH = 16384
EMB_DIM = 32
LANES = 16
NUSERS = 1000000
HB = 4
NBUF = 2

_info = plsc.get_sparse_core_info()
_NC, _NS = _info.num_cores, _info.num_subcores
NW = _NC * _NS
B_PER_W = BATCH // NW
CHUNK = 128
N_CHUNKS = B_PER_W // CHUNK
N_GROUPS = B_PER_W // LANES
NBATCH = LANES // HB


def _body(user_hbm, item_hbm, uet_hbm, iet_hbm, ubias_hbm, ibias_hbm,
          out_hbm, uidx_v, iidx_v, ublk_v, iblk_v, ubias_v, ibias_v, out_v,
          semA, semB, bsem):
    sems = [semA, semB]
    wid = lax.axis_index("s") * _NC + lax.axis_index("c")
    base = wid * B_PER_W

    pltpu.sync_copy(user_hbm.at[pl.ds(base, B_PER_W)], uidx_v)
    pltpu.sync_copy(item_hbm.at[pl.ds(base, B_PER_W)], iidx_v)

    bias_handles = []
    for j in range(N_CHUNKS):
        sl = pl.ds(j * CHUNK, CHUNK)
        bias_handles.append(pltpu.async_copy(
            ubias_hbm.at[0].at[uidx_v.at[sl]], ubias_v.at[sl], bsem))
        bias_handles.append(pltpu.async_copy(
            ibias_hbm.at[0].at[iidx_v.at[sl]], ibias_v.at[sl], bsem))

    lane_ids = lax.iota(jnp.int32, LANES)
    hi01 = lane_ids // 8
    hi23 = hi01 + 2
    lo8 = lane_ids % 8

    def fire(uvec, ivec, k, buf):
        for b in range(HB):
            l = k * HB + b
            ub = pl.multiple_of((uvec[l] >> 7) * 128, 128)
            ib = pl.multiple_of((ivec[l] >> 7) * 128, 128)
            pltpu.async_copy(uet_hbm.at[:, :, pl.ds(ub, 128)],
                             ublk_v.at[buf, b], sems[buf])
            pltpu.async_copy(iet_hbm.at[:, :, pl.ds(ib, 128)],
                             iblk_v.at[buf, b], sems[buf])

    def drain_batch(buf):
        for b in range(HB):
            pltpu.make_async_copy(uet_hbm.at[:, :, pl.ds(0, 128)],
                                  ublk_v.at[0, b], sems[buf]).wait()
            pltpu.make_async_copy(iet_hbm.at[:, :, pl.ds(0, 128)],
                                  iblk_v.at[0, b], sems[buf]).wait()

    uvec0 = uidx_v[pl.ds(0, LANES)]
    ivec0 = iidx_v[pl.ds(0, LANES)]
    fire(uvec0, ivec0, 0, 0)

    def group(g, _):
        rb = g * LANES
        uvec = uidx_v[pl.ds(rb, LANES)]
        ivec = iidx_v[pl.ds(rb, LANES)]
        gn = jnp.minimum(g + 1, N_GROUPS - 1) * LANES
        unext = uidx_v[pl.ds(gn, LANES)]
        inext = iidx_v[pl.ds(gn, LANES)]
        dots = jnp.zeros((LANES,), jnp.float32)
        for k in range(NBATCH):
            buf = k % NBUF
            nbuf = (k + 1) % NBUF
            if k < NBATCH - 1:
                fire(uvec, ivec, k + 1, nbuf)
            else:
                @pl.when(g < N_GROUPS - 1)
                def _():
                    fire(unext, inext, 0, nbuf)
            drain_batch(buf)
            for b in range(HB):
                l = k * HB + b
                ucol = jnp.broadcast_to(uvec[l] & 127, (LANES,))
                icol = jnp.broadcast_to(ivec[l] & 127, (LANES,))
                bv = jnp.full((LANES,), b, dtype=jnp.int32)
                bufv = jnp.full((LANES,), buf, dtype=jnp.int32)
                u1 = plsc.load_gather(ublk_v, [bufv, bv, hi01, lo8, ucol])
                u2 = plsc.load_gather(ublk_v, [bufv, bv, hi23, lo8, ucol])
                i1 = plsc.load_gather(iblk_v, [bufv, bv, hi01, lo8, icol])
                i2 = plsc.load_gather(iblk_v, [bufv, bv, hi23, lo8, icol])
                s = jnp.sum(u1 * i1 + u2 * i2)
                dots = jnp.where(lane_ids == l, s, dots)
        out_v[pl.ds(rb, LANES)] = (dots + ubias_v[pl.ds(rb, LANES)]
                                   + ibias_v[pl.ds(rb, LANES)])
        return 0

    lax.fori_loop(0, N_GROUPS, group, 0)

    for h in bias_handles:
        h.wait()
    pltpu.sync_copy(out_v, out_hbm.at[pl.ds(base, B_PER_W)])


@jax.jit
def _run(user, item, uet, iet, ubias_t, ibias_t):
    mesh = plsc.VectorSubcoreMesh(core_axis_name="c", subcore_axis_name="s")
    return pl.kernel(
        _body,
        mesh=mesh,
        out_type=jax.ShapeDtypeStruct((BATCH,), jnp.float32),
        scratch_types=[
            pltpu.VMEM((B_PER_W,), jnp.int32),
            pltpu.VMEM((B_PER_W,), jnp.int32),
            pltpu.VMEM((NBUF, HB, 4, 8, 128), jnp.float32),
            pltpu.VMEM((NBUF, HB, 4, 8, 128), jnp.float32),
            pltpu.VMEM((B_PER_W,), jnp.float32),
            pltpu.VMEM((B_PER_W,), jnp.float32),
            pltpu.VMEM((B_PER_W,), jnp.float32),
            pltpu.SemaphoreType.DMA,
            pltpu.SemaphoreType.DMA,
            pltpu.SemaphoreType.DMA,
        ],
        compiler_params=pltpu.CompilerParams(
            needs_layout_passes=False, disable_bounds_checks=True),
    )(user, item, uet, iet, ubias_t, ibias_t)


def kernel(user, item, user_embedding, item_embedding, user_bias, item_bias):
    uet = user_embedding.T.reshape(4, 8, NUSERS)
    iet = item_embedding.T.reshape(4, 8, NUSERS)
    return _run(user.astype(jnp.int32), item.astype(jnp.int32), uet, iet,
                user_bias.T, item_bias.T)


# speedup vs baseline: 4.0404x; 1.1063x over previous
import functools

import jax
import jax.numpy as jnp
from jax import lax
from jax.experimental import pallas as pl
from jax.experimental.pallas import tpu as pltpu
from jax.experimental.pallas import tpu_sc as plsc

BATCH = 16384
EMB_DIM = 32
LANES = 16
NUSERS = 1000000
HB = 2
NBUF = 4
DEPTH = 3

_info = plsc.get_sparse_core_info()
_NC, _NS = _info.num_cores, _info.num_subcores
NW = _NC * _NS
B_PER_W = BATCH // NW
CHUNK = 128
N_CHUNKS = B_PER_W // CHUNK
N_GROUPS = B_PER_W // LANES
NBATCH = LANES // HB


def _body(user_hbm, item_hbm, uet_hbm, iet_hbm, ubias_hbm, ibias_hbm,
          out_hbm, uidx_v, iidx_v, ublk_v, iblk_v, ubias_v, ibias_v, out_v,
          semA, semB, semC, semD, bsem):
    sems = [semA, semB, semC, semD]
    wid = lax.axis_index("s") * _NC + lax.axis_index("c")
    base = wid * B_PER_W

    pltpu.sync_copy(user_hbm.at[pl.ds(base, B_PER_W)], uidx_v)
    pltpu.sync_copy(item_hbm.at[pl.ds(base, B_PER_W)], iidx_v)

    bias_handles = []
    for j in range(N_CHUNKS):
        sl = pl.ds(j * CHUNK, CHUNK)
        bias_handles.append(pltpu.async_copy(
            ubias_hbm.at[0].at[uidx_v.at[sl]], ubias_v.at[sl], bsem))
        bias_handles.append(pltpu.async_copy(
            ibias_hbm.at[0].at[iidx_v.at[sl]], ibias_v.at[sl], bsem))

    lane_ids = lax.iota(jnp.int32, LANES)
    hi01 = lane_ids // 8
    hi23 = hi01 + 2
    lo8 = lane_ids % 8

    def fire(uvec, ivec, k, buf):
        for b in range(HB):
            l = k * HB + b
            ub = pl.multiple_of((uvec[l] >> 7) * 128, 128)
            ib = pl.multiple_of((ivec[l] >> 7) * 128, 128)
            pltpu.async_copy(uet_hbm.at[:, :, pl.ds(ub, 128)],
                             ublk_v.at[buf, b], sems[buf])
            pltpu.async_copy(iet_hbm.at[:, :, pl.ds(ib, 128)],
                             iblk_v.at[buf, b], sems[buf])

    def drain_batch(buf):
        for b in range(HB):
            pltpu.make_async_copy(uet_hbm.at[:, :, pl.ds(0, 128)],
                                  ublk_v.at[0, b], sems[buf]).wait()
            pltpu.make_async_copy(iet_hbm.at[:, :, pl.ds(0, 128)],
                                  iblk_v.at[0, b], sems[buf]).wait()

    uvec0 = uidx_v[pl.ds(0, LANES)]
    ivec0 = iidx_v[pl.ds(0, LANES)]
    for k in range(DEPTH):
        fire(uvec0, ivec0, k, k % NBUF)

    def group(g, _):
        rb = g * LANES
        uvec = uidx_v[pl.ds(rb, LANES)]
        ivec = iidx_v[pl.ds(rb, LANES)]
        gn = jnp.minimum(g + 1, N_GROUPS - 1) * LANES
        unext = uidx_v[pl.ds(gn, LANES)]
        inext = iidx_v[pl.ds(gn, LANES)]
        dots = jnp.zeros((LANES,), jnp.float32)
        for k in range(NBATCH):
            buf = k % NBUF
            nbuf = (k + DEPTH) % NBUF
            if k < NBATCH - DEPTH:
                fire(uvec, ivec, k + DEPTH, nbuf)
            else:
                @pl.when(g < N_GROUPS - 1)
                def _():
                    fire(unext, inext, k + DEPTH - NBATCH, nbuf)
            drain_batch(buf)
            for b in range(HB):
                l = k * HB + b
                ucol = jnp.broadcast_to(uvec[l] & 127, (LANES,))
                icol = jnp.broadcast_to(ivec[l] & 127, (LANES,))
                bv = jnp.full((LANES,), b, dtype=jnp.int32)
                bufv = jnp.full((LANES,), buf, dtype=jnp.int32)
                u1 = plsc.load_gather(ublk_v, [bufv, bv, hi01, lo8, ucol])
                u2 = plsc.load_gather(ublk_v, [bufv, bv, hi23, lo8, ucol])
                i1 = plsc.load_gather(iblk_v, [bufv, bv, hi01, lo8, icol])
                i2 = plsc.load_gather(iblk_v, [bufv, bv, hi23, lo8, icol])
                s = jnp.sum(u1 * i1 + u2 * i2)
                dots = jnp.where(lane_ids == l, s, dots)
        out_v[pl.ds(rb, LANES)] = (dots + ubias_v[pl.ds(rb, LANES)]
                                   + ibias_v[pl.ds(rb, LANES)])
        return 0

    lax.fori_loop(0, N_GROUPS, group, 0)

    for h in bias_handles:
        h.wait()
    pltpu.sync_copy(out_v, out_hbm.at[pl.ds(base, B_PER_W)])


@jax.jit
def _run(user, item, uet, iet, ubias_t, ibias_t):
    mesh = plsc.VectorSubcoreMesh(core_axis_name="c", subcore_axis_name="s")
    return pl.kernel(
        _body,
        mesh=mesh,
        out_type=jax.ShapeDtypeStruct((BATCH,), jnp.float32),
        scratch_types=[
            pltpu.VMEM((B_PER_W,), jnp.int32),
            pltpu.VMEM((B_PER_W,), jnp.int32),
            pltpu.VMEM((NBUF, HB, 4, 8, 128), jnp.float32),
            pltpu.VMEM((NBUF, HB, 4, 8, 128), jnp.float32),
            pltpu.VMEM((B_PER_W,), jnp.float32),
            pltpu.VMEM((B_PER_W,), jnp.float32),
            pltpu.VMEM((B_PER_W,), jnp.float32),
            pltpu.SemaphoreType.DMA,
            pltpu.SemaphoreType.DMA,
            pltpu.SemaphoreType.DMA,
            pltpu.SemaphoreType.DMA,
            pltpu.SemaphoreType.DMA,
        ],
        compiler_params=pltpu.CompilerParams(
            needs_layout_passes=False, disable_bounds_checks=True),
    )(user, item, uet, iet, ubias_t, ibias_t)


def kernel(user, item, user_embedding, item_embedding, user_bias, item_bias):
    uet = user_embedding.T.reshape(4, 8, NUSERS)
    iet = item_embedding.T.reshape(4, 8, NUSERS)
    return _run(user.astype(jnp.int32), item.astype(jnp.int32), uet, iet,
                user_bias.T, item_bias.T)


# speedup vs baseline: 4.2360x; 1.0484x over previous
import functools

import jax
import jax.numpy as jnp
from jax import lax
from jax.experimental import pallas as pl
from jax.experimental.pallas import tpu as pltpu
from jax.experimental.pallas import tpu_sc as plsc

BATCH = 16384
EMB_DIM = 32
LANES = 16
NUSERS = 1000000
HB = 1
NBUF = 8
DEPTH = 6

_info = plsc.get_sparse_core_info()
_NC, _NS = _info.num_cores, _info.num_subcores
NW = _NC * _NS
B_PER_W = BATCH // NW
CHUNK = 128
N_CHUNKS = B_PER_W // CHUNK
N_GROUPS = B_PER_W // LANES
NBATCH = LANES // HB


def _body(user_hbm, item_hbm, uet_hbm, iet_hbm, ubias_hbm, ibias_hbm,
          out_hbm, uidx_v, iidx_v, ublk_v, iblk_v, ubias_v, ibias_v, out_v,
          semA, semB, semC, semD, semE, semF, semG, semH, bsem):
    sems = [semA, semB, semC, semD, semE, semF, semG, semH]
    wid = lax.axis_index("s") * _NC + lax.axis_index("c")
    base = wid * B_PER_W

    pltpu.sync_copy(user_hbm.at[pl.ds(base, B_PER_W)], uidx_v)
    pltpu.sync_copy(item_hbm.at[pl.ds(base, B_PER_W)], iidx_v)

    bias_handles = []
    for j in range(N_CHUNKS):
        sl = pl.ds(j * CHUNK, CHUNK)
        bias_handles.append(pltpu.async_copy(
            ubias_hbm.at[0].at[uidx_v.at[sl]], ubias_v.at[sl], bsem))
        bias_handles.append(pltpu.async_copy(
            ibias_hbm.at[0].at[iidx_v.at[sl]], ibias_v.at[sl], bsem))

    lane_ids = lax.iota(jnp.int32, LANES)
    hi01 = lane_ids // 8
    hi23 = hi01 + 2
    lo8 = lane_ids % 8

    def fire(uvec, ivec, k, buf):
        for b in range(HB):
            l = k * HB + b
            ub = pl.multiple_of((uvec[l] >> 7) * 128, 128)
            ib = pl.multiple_of((ivec[l] >> 7) * 128, 128)
            pltpu.async_copy(uet_hbm.at[:, :, pl.ds(ub, 128)],
                             ublk_v.at[buf, b], sems[buf])
            pltpu.async_copy(iet_hbm.at[:, :, pl.ds(ib, 128)],
                             iblk_v.at[buf, b], sems[buf])

    def drain_batch(buf):
        for b in range(HB):
            pltpu.make_async_copy(uet_hbm.at[:, :, pl.ds(0, 128)],
                                  ublk_v.at[0, b], sems[buf]).wait()
            pltpu.make_async_copy(iet_hbm.at[:, :, pl.ds(0, 128)],
                                  iblk_v.at[0, b], sems[buf]).wait()

    uvec0 = uidx_v[pl.ds(0, LANES)]
    ivec0 = iidx_v[pl.ds(0, LANES)]
    for k in range(DEPTH):
        fire(uvec0, ivec0, k, k % NBUF)

    def group(g, _):
        rb = g * LANES
        uvec = uidx_v[pl.ds(rb, LANES)]
        ivec = iidx_v[pl.ds(rb, LANES)]
        gn = jnp.minimum(g + 1, N_GROUPS - 1) * LANES
        unext = uidx_v[pl.ds(gn, LANES)]
        inext = iidx_v[pl.ds(gn, LANES)]
        dots = jnp.zeros((LANES,), jnp.float32)
        for k in range(NBATCH):
            buf = k % NBUF
            nbuf = (k + DEPTH) % NBUF
            if k < NBATCH - DEPTH:
                fire(uvec, ivec, k + DEPTH, nbuf)
            else:
                @pl.when(g < N_GROUPS - 1)
                def _():
                    fire(unext, inext, k + DEPTH - NBATCH, nbuf)
            drain_batch(buf)
            for b in range(HB):
                l = k * HB + b
                ucol = jnp.broadcast_to(uvec[l] & 127, (LANES,))
                icol = jnp.broadcast_to(ivec[l] & 127, (LANES,))
                bv = jnp.full((LANES,), b, dtype=jnp.int32)
                bufv = jnp.full((LANES,), buf, dtype=jnp.int32)
                u1 = plsc.load_gather(ublk_v, [bufv, bv, hi01, lo8, ucol])
                u2 = plsc.load_gather(ublk_v, [bufv, bv, hi23, lo8, ucol])
                i1 = plsc.load_gather(iblk_v, [bufv, bv, hi01, lo8, icol])
                i2 = plsc.load_gather(iblk_v, [bufv, bv, hi23, lo8, icol])
                s = jnp.sum(u1 * i1 + u2 * i2)
                dots = jnp.where(lane_ids == l, s, dots)
        out_v[pl.ds(rb, LANES)] = (dots + ubias_v[pl.ds(rb, LANES)]
                                   + ibias_v[pl.ds(rb, LANES)])
        return 0

    lax.fori_loop(0, N_GROUPS, group, 0)

    for h in bias_handles:
        h.wait()
    pltpu.sync_copy(out_v, out_hbm.at[pl.ds(base, B_PER_W)])


@jax.jit
def _run(user, item, uet, iet, ubias_t, ibias_t):
    mesh = plsc.VectorSubcoreMesh(core_axis_name="c", subcore_axis_name="s")
    return pl.kernel(
        _body,
        mesh=mesh,
        out_type=jax.ShapeDtypeStruct((BATCH,), jnp.float32),
        scratch_types=[
            pltpu.VMEM((B_PER_W,), jnp.int32),
            pltpu.VMEM((B_PER_W,), jnp.int32),
            pltpu.VMEM((NBUF, HB, 4, 8, 128), jnp.float32),
            pltpu.VMEM((NBUF, HB, 4, 8, 128), jnp.float32),
            pltpu.VMEM((B_PER_W,), jnp.float32),
            pltpu.VMEM((B_PER_W,), jnp.float32),
            pltpu.VMEM((B_PER_W,), jnp.float32),
            pltpu.SemaphoreType.DMA,
            pltpu.SemaphoreType.DMA,
            pltpu.SemaphoreType.DMA,
            pltpu.SemaphoreType.DMA,
            pltpu.SemaphoreType.DMA,
            pltpu.SemaphoreType.DMA,
            pltpu.SemaphoreType.DMA,
            pltpu.SemaphoreType.DMA,
            pltpu.SemaphoreType.DMA,
        ],
        compiler_params=pltpu.CompilerParams(
            needs_layout_passes=False, disable_bounds_checks=True),
    )(user, item, uet, iet, ubias_t, ibias_t)


def kernel(user, item, user_embedding, item_embedding, user_bias, item_bias):
    uet = user_embedding.T.reshape(4, 8, NUSERS)
    iet = item_embedding.T.reshape(4, 8, NUSERS)
    return _run(user.astype(jnp.int32), item.astype(jnp.int32), uet, iet,
                user_bias.T, item_bias.T)
